# Initial kernel scaffold; baseline (speedup 1.0000x reference)
import jax
import jax.numpy as jnp
from jax.experimental import pallas as pl


def kernel(x, edge_index, W0_0, W0_1, W0_2, b0, W1_0, W1_1, W1_2, b1):
    raise NotImplementedError("write your pallas kernel here")


# speedup vs baseline: 1.7022x; 1.7022x over previous
import functools
import jax
import jax.numpy as jnp
from jax.experimental import pallas as pl
from jax.experimental.pallas import tpu as pltpu

N_NODES = 50000
BM = 1000


def _layer_body(h_ref, s1_ref, s2_ref, dinv_ref, w0_ref, w1_ref, w2_ref,
                b_ref, out_ref):
    dinv = dinv_ref[...]
    h = h_ref[...]
    p1 = dinv * s1_ref[...]
    p2 = dinv * s2_ref[...]
    acc = jax.lax.dot_general(h, w0_ref[...], (((1,), (1,)), ((), ())),
                              preferred_element_type=jnp.float32)
    acc += jax.lax.dot_general(p1, w1_ref[...], (((1,), (1,)), ((), ())),
                               preferred_element_type=jnp.float32)
    acc += jax.lax.dot_general(p2, w2_ref[...], (((1,), (1,)), ((), ())),
                               preferred_element_type=jnp.float32)
    out_ref[...] = jnp.maximum(acc + b_ref[...], 0.0)


def _tag_layer(h, s1, s2, dinv, w0, w1, w2, b):
    n, d_in = h.shape
    d_out = w0.shape[0]
    grid = (n // BM,)
    row_spec = lambda: pl.BlockSpec((BM, d_in), lambda i: (i, 0))
    full = lambda r, c: pl.BlockSpec((r, c), lambda i: (0, 0))
    return pl.pallas_call(
        _layer_body,
        grid=grid,
        in_specs=[
            row_spec(), row_spec(), row_spec(),
            pl.BlockSpec((BM, 1), lambda i: (i, 0)),
            full(d_out, d_in), full(d_out, d_in), full(d_out, d_in),
            full(1, d_out),
        ],
        out_specs=pl.BlockSpec((BM, d_out), lambda i: (i, 0)),
        out_shape=jax.ShapeDtypeStruct((n, d_out), jnp.float32),
    )(h, s1, s2, dinv, w0, w1, w2, b)


def _scatter_sum(u, row, col):
    msg = u[row]
    return jnp.zeros((N_NODES, u.shape[1]), jnp.float32).at[col].add(msg)


def kernel(x, edge_index, W0_0, W0_1, W0_2, b0, W1_0, W1_1, W1_2, b1):
    row = edge_index[0].astype(jnp.int32)
    col = edge_index[1].astype(jnp.int32)

    deg = jnp.zeros((N_NODES,), jnp.float32).at[col].add(1.0)
    dinv = jnp.where(deg > 0, jax.lax.rsqrt(deg), 0.0)[:, None]

    u1 = dinv * x
    s1 = _scatter_sum(u1, row, col)
    s2 = _scatter_sum(dinv * dinv * s1, row, col)
    h = _tag_layer(x, s1, s2, dinv, W0_0, W0_1, W0_2, b0[None, :])

    t1 = _scatter_sum(dinv * h, row, col)
    t2 = _scatter_sum(dinv * dinv * t1, row, col)
    out = _tag_layer(h, t1, t2, dinv, W1_0, W1_1, W1_2, b1[None, :])
    return out
